# Initial kernel scaffold; baseline (speedup 1.0000x reference)
import jax
import jax.numpy as jnp
from jax.experimental import pallas as pl


def kernel(x, edge_index, edge_attr, batch, x_emb1, x_emb2, mlp_W1, mlp_b1, mlp_W2, mlp_b2, e_emb1, e_emb2, e_W3, e_b3, bn_gamma, bn_beta, feat_W, feat_b, p_W1, p_b1, p_W2, p_b2, p_W3, p_b3):
    raise NotImplementedError("write your pallas kernel here")


# speedup vs baseline: 4.0654x; 4.0654x over previous
import functools

import jax
import jax.numpy as jnp
from jax import lax
from jax.experimental import pallas as pl
from jax.experimental.pallas import tpu as pltpu
from jax.experimental.pallas import tpu_sc as plsc

N = 10000
NPAD = 10240
E_REAL = 320000
D = 128
FW = 128
NSUB = 16
NCORE = 2
NW = NCORE * NSUB
EPW = 10240
E_PAD = NW * EPW
CHUNK = 128
NLAYER = 5
NGRAPH = 512
FEAT = 512

_f32 = jnp.float32
QSCALE = float(2 ** 21)
QINV = float(2.0 ** -21)


def _bf(v):
    return v.astype(jnp.bfloat16)


@functools.lru_cache(maxsize=None)
def _sc_kernels():
    mesh = plsc.VectorSubcoreMesh(core_axis_name="c", subcore_axis_name="s")

    @functools.partial(
        pl.kernel, mesh=mesh,
        out_type=jax.ShapeDtypeStruct((NCORE, NPAD, D), jnp.int32),
        scratch_types=[
            pltpu.VMEM((CHUNK,), jnp.int32),
            pltpu.VMEM((CHUNK,), jnp.int32),
            pltpu.VMEM((CHUNK, D), jnp.int32),
            pltpu.VMEM_SHARED((NPAD, D), jnp.int32),
            pltpu.SemaphoreType.DMA,
        ],
    )
    def sc_gather_scatter(h_hbm, src_hbm, dst_hbm, z_hbm, out_hbm,
                          src_v, dst_v, rows_v, agg_s, sem):
        cid = lax.axis_index("c")
        sid = lax.axis_index("s")
        base = (cid * NSUB + sid) * EPW
        stripe = NPAD // NSUB
        row0 = sid * stripe
        pltpu.sync_copy(z_hbm.at[pl.ds(row0, stripe)],
                        agg_s.at[pl.ds(row0, stripe)])
        plsc.subcore_barrier()

        @pl.loop(0, EPW // CHUNK)
        def _(g):
            off = base + g * CHUNK
            pltpu.sync_copy(src_hbm.at[pl.ds(off, CHUNK)], src_v)
            pltpu.async_copy(h_hbm.at[src_v], rows_v, sem).wait()
            pltpu.sync_copy(dst_hbm.at[pl.ds(off, CHUNK)], dst_v)
            pltpu.sync_copy(rows_v, agg_s.at[dst_v], add=True)

        plsc.subcore_barrier()
        pltpu.sync_copy(agg_s.at[pl.ds(row0, stripe)],
                        out_hbm.at[cid, pl.ds(row0, stripe)])

    @functools.partial(
        pl.kernel, mesh=mesh,
        out_type=jax.ShapeDtypeStruct((NCORE, NPAD, FW), _f32),
        scratch_types=[
            pltpu.VMEM((CHUNK,), jnp.int32),
            pltpu.VMEM((CHUNK, FW), _f32),
            pltpu.VMEM_SHARED((NPAD, FW), _f32),
        ],
    )
    def sc_linear_scatter(feat_hbm, dst_hbm, z_hbm, out_hbm,
                          dst_v, feat_v, agg_s):
        cid = lax.axis_index("c")
        sid = lax.axis_index("s")
        base = (cid * NSUB + sid) * EPW
        stripe = NPAD // NSUB
        row0 = sid * stripe
        pltpu.sync_copy(z_hbm.at[pl.ds(row0, stripe)],
                        agg_s.at[pl.ds(row0, stripe)])
        plsc.subcore_barrier()

        @pl.loop(0, EPW // CHUNK)
        def _(g):
            off = base + g * CHUNK
            pltpu.sync_copy(feat_hbm.at[pl.ds(off, CHUNK)], feat_v)
            pltpu.sync_copy(dst_hbm.at[pl.ds(off, CHUNK)], dst_v)
            pltpu.sync_copy(feat_v, agg_s.at[dst_v], add=True)

        plsc.subcore_barrier()
        pltpu.sync_copy(agg_s.at[pl.ds(row0, stripe)],
                        out_hbm.at[cid, pl.ds(row0, stripe)])

    return sc_gather_scatter, sc_linear_scatter


EB = 4096


def _e19_body(ea_ref, o_ref):
    a = ea_ref[...].astype(_f32)
    j = lax.broadcasted_iota(jnp.int32, (EB, FW), 1)
    ea0 = ea_ref[:, 0:1]
    ea1 = ea_ref[:, 1:2]
    out = jnp.zeros((EB, FW), _f32)
    for k in range(8):
        out = out + jnp.where(j == k, a[:, 2 + k:3 + k], 0.0)
    out = out + jnp.where((j >= 8) & (j < 14) & (j - 8 == ea0), 1.0, 0.0)
    out = out + jnp.where((j >= 14) & (j < 18) & (j - 14 == ea1), 1.0, 0.0)
    out = out + jnp.where(j == 18, 1.0, 0.0)
    o_ref[...] = out


def _build_e19(ea_pad):
    return pl.pallas_call(
        _e19_body,
        grid=(E_PAD // EB,),
        in_specs=[pl.BlockSpec((EB, 10), lambda i: (i, 0))],
        out_specs=pl.BlockSpec((EB, FW), lambda i: (i, 0)),
        out_shape=jax.ShapeDtypeStruct((E_PAD, FW), _f32),
    )(ea_pad)


def _h0_body(x_ref, e1_ref, e2_ref, o_ref, q_ref):
    x0 = x_ref[:, 0:1]
    x1 = x_ref[:, 1:2]
    j1 = lax.broadcasted_iota(jnp.int32, (N, 120), 1)
    j2 = lax.broadcasted_iota(jnp.int32, (N, 8), 1)
    oh1 = (j1 == x0).astype(_f32)
    oh2 = ((j2 == x1) & (j2 < 3)).astype(_f32)
    h0 = lax.dot_general(oh1, e1_ref[...], (((1,), (0,)), ((), ())),
                         preferred_element_type=_f32, precision=lax.Precision.HIGHEST)
    h0 = h0 + lax.dot_general(oh2, e2_ref[...], (((1,), (0,)), ((), ())),
                              preferred_element_type=_f32, precision=lax.Precision.HIGHEST)
    o_ref[...] = h0
    q_ref[...] = jnp.rint(h0 * QSCALE).astype(jnp.int32)


def _build_h0(x, x_emb1, x_emb2pad):
    return pl.pallas_call(
        _h0_body,
        out_shape=(jax.ShapeDtypeStruct((N, D), _f32),
                   jax.ShapeDtypeStruct((N, D), jnp.int32)),
    )(x, x_emb1, x_emb2pad)


def _dense_body(aggp_ref, cp_ref, h_ref, t_ref, const_ref,
                w1_ref, b1_ref, w2_ref, b2_ref, g_ref, be_ref, o_ref, q_ref, *, last):
    agg = (aggp_ref[0] + aggp_ref[1]).astype(_f32) * QINV + h_ref[...]
    c = cp_ref[0] + cp_ref[1]
    agg = agg + lax.dot_general(_bf(c[:, :8]), _bf(t_ref[0:8]),
                                (((1,), (0,)), ((), ())),
                                preferred_element_type=_f32)
    agg = agg + lax.dot_general(c[:, 8:], t_ref[8:], (((1,), (0,)), ((), ())),
                                preferred_element_type=_f32, precision=lax.Precision.HIGHEST)
    agg = agg + const_ref[...]
    hmid = lax.dot_general(_bf(agg), _bf(w1_ref[...]), (((1,), (1,)), ((), ())),
                           preferred_element_type=_f32) + b1_ref[...]
    hmid = jnp.maximum(hmid, 0.0)
    h2 = lax.dot_general(_bf(hmid), _bf(w2_ref[...]), (((1,), (1,)), ((), ())),
                         preferred_element_type=_f32) + b2_ref[...]
    mu = jnp.mean(h2, axis=0, keepdims=True)
    cen = h2 - mu
    var = jnp.mean(cen * cen, axis=0, keepdims=True)
    h2 = cen / jnp.sqrt(var + 1e-5) * g_ref[...] + be_ref[...]
    if not last:
        h2 = jnp.maximum(h2, 0.0)
    o_ref[...] = h2
    q_ref[...] = jnp.rint(h2 * QSCALE).astype(jnp.int32)


def _dense_layer(aggp, cp, h, t, const, w1, b1, w2, b2, gamma, beta, last):
    return pl.pallas_call(
        functools.partial(_dense_body, last=last),
        out_shape=(jax.ShapeDtypeStruct((N, D), _f32),
                   jax.ShapeDtypeStruct((N, D), jnp.int32)),
    )(aggp, cp, h, t, const, w1, b1, w2, b2, gamma, beta)


PB = 1000
PGRID = N // PB


def _final_body(h_ref, b_ref, fw_ref, fb_ref, w1_ref, b1_ref, w2_ref, b2_ref,
                w3_ref, b3_ref, o_ref, pool_acc, cnt_acc):
    i = pl.program_id(0)

    @pl.when(i == 0)
    def _():
        pool_acc[...] = jnp.zeros((NGRAPH, D), _f32)
        cnt_acc[...] = jnp.zeros((NGRAPH, 8), _f32)

    j = lax.broadcasted_iota(jnp.int32, (PB, NGRAPH), 1)
    oh = (j == b_ref[...]).astype(_f32)
    pool_acc[...] += lax.dot_general(oh, h_ref[...], (((0,), (0,)), ((), ())),
                                     preferred_element_type=_f32, precision=lax.Precision.HIGHEST)
    cnt_acc[...] += lax.dot_general(oh, jnp.ones((PB, 8), _f32),
                                    (((0,), (0,)), ((), ())),
                                    preferred_element_type=_f32, precision=lax.Precision.HIGHEST)

    @pl.when(i == PGRID - 1)
    def _():
        cnt = jnp.maximum(cnt_acc[:, 0:1], 1.0)
        pooled = pool_acc[...] / cnt
        hf = lax.dot_general(_bf(pooled), _bf(fw_ref[...]), (((1,), (1,)), ((), ())),
                             preferred_element_type=_f32) + fb_ref[...]
        z = lax.dot_general(_bf(hf), _bf(w1_ref[...]), (((1,), (1,)), ((), ())),
                            preferred_element_type=_f32) + b1_ref[...]
        p = jnp.maximum(z, 0.0) + jnp.log(1.0 + jnp.exp(-jnp.abs(z)))
        z = lax.dot_general(_bf(p), _bf(w2_ref[...]), (((1,), (1,)), ((), ())),
                            preferred_element_type=_f32) + b2_ref[...]
        p = jnp.maximum(z, 0.0) + jnp.log(1.0 + jnp.exp(-jnp.abs(z)))
        o_ref[...] = lax.dot_general(_bf(p), _bf(w3_ref[...]), (((1,), (1,)), ((), ())),
                                     preferred_element_type=_f32) + b3_ref[...]


def _final(h_node, batch2d, feat_W, feat_b, p_W1, p_b1, p_W2, p_b2,
           p_W3pad, p_b3pad):
    full = lambda shp: pl.BlockSpec(shp, lambda i: tuple(0 for _ in shp))
    return pl.pallas_call(
        _final_body,
        grid=(PGRID,),
        in_specs=[
            pl.BlockSpec((PB, D), lambda i: (i, 0)),
            pl.BlockSpec((PB, 1), lambda i: (i, 0)),
            full((FEAT, D)), full((1, FEAT)),
            full((FEAT // 2, FEAT)), full((1, FEAT // 2)),
            full((FEAT // 2, FEAT // 2)), full((1, FEAT // 2)),
            full((8, FEAT // 2)), full((1, 8)),
        ],
        out_specs=pl.BlockSpec((NGRAPH, 8), lambda i: (0, 0)),
        out_shape=jax.ShapeDtypeStruct((NGRAPH, 8), _f32),
        scratch_shapes=[pltpu.VMEM((NGRAPH, D), _f32),
                        pltpu.VMEM((NGRAPH, 8), _f32)],
    )(h_node, batch2d, feat_W, feat_b, p_W1, p_b1, p_W2, p_b2, p_W3pad, p_b3pad)


def kernel(x, edge_index, edge_attr, batch,
           x_emb1, x_emb2, mlp_W1, mlp_b1, mlp_W2, mlp_b2,
           e_emb1, e_emb2, e_W3, e_b3, bn_gamma, bn_beta,
           feat_W, feat_b, p_W1, p_b1, p_W2, p_b2, p_W3, p_b3):
    npad_e = E_PAD - E_REAL
    src = jnp.concatenate([edge_index[0].astype(jnp.int32),
                           jnp.zeros((npad_e,), jnp.int32)])
    dst = jnp.concatenate([edge_index[1].astype(jnp.int32),
                           jnp.full((npad_e,), N, jnp.int32)])
    ea_pad = jnp.concatenate(
        [edge_attr.astype(jnp.int32), jnp.zeros((npad_e, 10), jnp.int32)], axis=0)
    z128 = jnp.zeros((NPAD, D), jnp.int32)
    z32 = jnp.zeros((NPAD, FW), _f32)

    t_all = jnp.concatenate([
        jnp.swapaxes(e_W3, 1, 2),
        e_emb1,
        e_emb2,
        e_b3[:, None, :],
        jnp.zeros((NLAYER, FW - 19, D), _f32)], axis=1)
    const_all = e_emb1[:, 4, :] + e_emb2[:, 0, :] + e_b3

    x_emb2pad = jnp.concatenate([x_emb2.astype(_f32),
                                 jnp.zeros((5, D), _f32)], axis=0)

    sc_gather_scatter, sc_linear_scatter = _sc_kernels()
    e19 = _build_e19(ea_pad)
    cp = sc_linear_scatter(e19, dst, z32)[:, :N, :]
    h, q = _build_h0(x.astype(jnp.int32), x_emb1.astype(_f32), x_emb2pad)

    for l in range(NLAYER):
        aggp = sc_gather_scatter(q, src, dst, z128)[:, :N, :]
        h, q = _dense_layer(aggp, cp, h,
                         t_all[l], const_all[l][None, :],
                         mlp_W1[l], mlp_b1[l][None, :],
                         mlp_W2[l], mlp_b2[l][None, :],
                         bn_gamma[l][None, :], bn_beta[l][None, :],
                         last=(l == NLAYER - 1))

    pred = _final(h, batch.astype(jnp.int32).reshape(N, 1),
                  feat_W, feat_b[None, :], p_W1, p_b1[None, :],
                  p_W2, p_b2[None, :],
                  jnp.concatenate([p_W3, jnp.zeros((7, FEAT // 2), _f32)], axis=0),
                  jnp.concatenate([p_b3, jnp.zeros((7,), _f32)])[None, :])
    return (h, pred[:, :1])
